# baseline (speedup 1.0000x reference)
import functools

import jax
import jax.numpy as jnp
from jax import lax
from jax.experimental import pallas as pl
from jax.experimental.pallas import tpu as pltpu
from jax.experimental.pallas import tpu_sc as plsc

_B, _D, _K, _A, _H = 4096, 256, 1024, 32, 64
_BLK = 512
_NBLK = _B // _BLK

_NC, _NS = 2, 16
_NW = _NC * _NS
_BPW = _B // _NW
_HP = 128


def _dist_argmin_body(x_ref, cbT_ref, cb_ref, W1_ref, b1_ref,
                      idx_ref, lsum_ref, cbw1_ref):
    i = pl.program_id(0)
    x = x_ref[...]
    cbT = cbT_ref[...]
    xdotc = jnp.dot(x, cbT, preferred_element_type=jnp.float32)
    xn = jnp.sum(x * x, axis=1, keepdims=True)
    cbn = jnp.sum(cbT * cbT, axis=0, keepdims=True)
    d = xn + cbn - 2.0 * xdotc
    minval = jnp.min(d, axis=1, keepdims=True)
    ids = lax.broadcasted_iota(jnp.int32, d.shape, 1)
    idx = jnp.min(jnp.where(d == minval, ids, _K), axis=1, keepdims=True)
    idx_ref[...] = idx

    @pl.when(i == 0)
    def _():
        lsum_ref[...] = jnp.zeros((1, 1), jnp.float32)
        cbw1_ref[...] = (jnp.dot(cb_ref[...], W1_ref[...],
                                 preferred_element_type=jnp.float32)
                         + b1_ref[...])

    lsum_ref[...] += jnp.sum(minval, axis=0, keepdims=True)


_dist_argmin = pl.pallas_call(
    _dist_argmin_body,
    grid=(_NBLK,),
    in_specs=[
        pl.BlockSpec((_BLK, _D), lambda i: (i, 0)),
        pl.BlockSpec((_D, _K), lambda i: (0, 0)),
        pl.BlockSpec((_K, _D), lambda i: (0, 0)),
        pl.BlockSpec((_D, _HP), lambda i: (0, 0)),
        pl.BlockSpec((1, _HP), lambda i: (0, 0)),
    ],
    out_specs=[
        pl.BlockSpec((_BLK, 1), lambda i: (i, 0)),
        pl.BlockSpec((1, 1), lambda i: (0, 0)),
        pl.BlockSpec((_K, _HP), lambda i: (0, 0)),
    ],
    out_shape=[
        jax.ShapeDtypeStruct((_B, 1), jnp.int32),
        jax.ShapeDtypeStruct((1, 1), jnp.float32),
        jax.ShapeDtypeStruct((_K, _HP), jnp.float32),
    ],
)


@functools.cache
def _make_sc_gather():
    @functools.partial(
        pl.kernel,
        mesh=plsc.VectorSubcoreMesh(core_axis_name="c", subcore_axis_name="s",
                                    num_cores=_NC, num_subcores=_NS),
        out_type=jax.ShapeDtypeStruct((_B, _HP), jnp.float32),
        scratch_types=[
            pltpu.VMEM((_BPW,), jnp.int32),
            pltpu.VMEM((_BPW, _HP), jnp.float32),
            pltpu.SemaphoreType.DMA,
        ],
        compiler_params=pltpu.CompilerParams(use_tc_tiling_on_sc=True),
    )
    def _sc_gather(table_hbm, idx_hbm, out_hbm, idx_v, rows_v, sem):
        wid = lax.axis_index("s") * _NC + lax.axis_index("c")
        base = wid * _BPW
        pltpu.sync_copy(idx_hbm.at[pl.ds(base, _BPW)], idx_v)
        pltpu.async_copy(table_hbm.at[idx_v], rows_v, sem).wait()
        pltpu.sync_copy(rows_v, out_hbm.at[pl.ds(base, _BPW)])

    return _sc_gather


def _mlp_body(g_ref, W2_ref, b2_ref, W3_ref, b3_ref, lsum_ref,
              out_ref, loss_ref):
    h1 = jnp.tanh(g_ref[...])
    h2 = jnp.tanh(jnp.dot(h1, W2_ref[...],
                          preferred_element_type=jnp.float32) + b2_ref[...])
    out_ref[...] = (jnp.dot(h2, W3_ref[...],
                            preferred_element_type=jnp.float32) + b3_ref[...])
    loss_ref[...] = lsum_ref[...] * (1.0 / (_B * _D))


_mlp = pl.pallas_call(
    _mlp_body,
    out_shape=[
        jax.ShapeDtypeStruct((_B, _A), jnp.float32),
        jax.ShapeDtypeStruct((1, 1), jnp.float32),
    ],
)


@jax.jit
def kernel(x, codebook, W1, b1, W2, b2, W3, b3):
    cbT = codebook.T
    W1p = jnp.pad(W1, ((0, 0), (0, _HP - _H)))
    b1p = jnp.pad(b1, (0, _HP - _H)).reshape(1, _HP)
    W2p = jnp.pad(W2, ((0, _HP - _H), (0, 0)))
    idx2d, lsum, cbw1b = _dist_argmin(x, cbT, codebook, W1p, b1p)
    idx = idx2d.reshape(_B)
    g = _make_sc_gather()(cbw1b, idx)
    dist, loss = _mlp(g, W2p, b2.reshape(1, _H), W3, b3.reshape(1, _A), lsum)
    return dist, loss.reshape(())


# speedup vs baseline: 1.9196x; 1.9196x over previous
import jax
import jax.numpy as jnp
from jax import lax
from jax.experimental import pallas as pl
from jax.experimental.pallas import tpu as pltpu

_B, _D, _K, _A, _H = 4096, 256, 1024, 32, 64
_BLK = 512
_NBLK = _B // _BLK


def _fused_body(x_ref, cbT_ref, cb_ref, W1_ref, b1_ref, W2_ref, b2_ref,
                W3_ref, b3_ref, out_ref, loss_ref, cbw1_scr):
    i = pl.program_id(0)

    @pl.when(i == 0)
    def _():
        cbw1_scr[...] = (jnp.dot(cb_ref[...], W1_ref[...],
                                 preferred_element_type=jnp.float32)
                         + b1_ref[...])
        loss_ref[...] = jnp.zeros((1, 1), jnp.float32)

    x = x_ref[...]
    xdotc = jnp.dot(x, cbT_ref[...], preferred_element_type=jnp.float32)
    xn = jnp.sum(x * x, axis=1, keepdims=True)
    cbn = jnp.sum(cbT_ref[...] * cbT_ref[...], axis=0, keepdims=True)
    d = xn + cbn - 2.0 * xdotc
    minval = jnp.min(d, axis=1, keepdims=True)
    ids = lax.broadcasted_iota(jnp.int32, d.shape, 1)
    idx = jnp.min(jnp.where(d == minval, ids, _K), axis=1, keepdims=True)
    onehot = (ids == idx).astype(jnp.float32)
    h1 = jnp.tanh(jnp.dot(onehot, cbw1_scr[...],
                          preferred_element_type=jnp.float32))
    h2 = jnp.tanh(jnp.dot(h1, W2_ref[...],
                          preferred_element_type=jnp.float32) + b2_ref[...])
    out_ref[...] = (jnp.dot(h2, W3_ref[...],
                            preferred_element_type=jnp.float32) + b3_ref[...])
    loss_ref[...] += jnp.sum(minval, axis=0, keepdims=True) * (1.0 / (_B * _D))


_fused = pl.pallas_call(
    _fused_body,
    grid=(_NBLK,),
    in_specs=[
        pl.BlockSpec((_BLK, _D), lambda i: (i, 0)),
        pl.BlockSpec((_D, _K), lambda i: (0, 0)),
        pl.BlockSpec((_K, _D), lambda i: (0, 0)),
        pl.BlockSpec((_D, _H), lambda i: (0, 0)),
        pl.BlockSpec((1, _H), lambda i: (0, 0)),
        pl.BlockSpec((_H, _H), lambda i: (0, 0)),
        pl.BlockSpec((1, _H), lambda i: (0, 0)),
        pl.BlockSpec((_H, _A), lambda i: (0, 0)),
        pl.BlockSpec((1, _A), lambda i: (0, 0)),
    ],
    out_specs=[
        pl.BlockSpec((_BLK, _A), lambda i: (i, 0)),
        pl.BlockSpec((1, 1), lambda i: (0, 0)),
    ],
    out_shape=[
        jax.ShapeDtypeStruct((_B, _A), jnp.float32),
        jax.ShapeDtypeStruct((1, 1), jnp.float32),
    ],
    scratch_shapes=[pltpu.VMEM((_K, _H), jnp.float32)],
)


@jax.jit
def kernel(x, codebook, W1, b1, W2, b2, W3, b3):
    dist, loss = _fused(x, codebook.T, codebook, W1, b1.reshape(1, _H),
                        W2, b2.reshape(1, _H), W3, b3.reshape(1, _A))
    return dist, loss.reshape(())
